# Initial kernel scaffold; baseline (speedup 1.0000x reference)
import functools

import jax
import jax.numpy as jnp
from jax import lax
from jax.experimental import pallas as pl
from jax.experimental.pallas import tpu as pltpu
from jax.experimental.pallas import tpu_sc as plsc

VOCAB = 1000000
DIM = 64
B = 16384
L = 20

NC = 2
NS = 16
NW = NC * NS

BW = B // NW
CB = 32
NCHUNK = BW // CB
RW = BW * L
CR = CB * L
GSUB = CR // 128


def _sc_kernel(eids_hbm, rids_hbm, fids_hbm, etab_hbm, rtab_hbm, ftab_hbm,
               eout_hbm, rout_hbm, fout_hbm,
               eidx_v, ridx_v, fidx_v, ebuf, rbuf, fbuf, sem):
    wid = lax.axis_index("s") * NC + lax.axis_index("c")
    rbase = wid * RW
    bbase = wid * BW

    pltpu.sync_copy(eids_hbm.at[pl.ds(rbase, RW)], eidx_v)
    pltpu.sync_copy(rids_hbm.at[pl.ds(rbase, RW)], ridx_v)
    pltpu.sync_copy(fids_hbm.at[pl.ds(bbase, BW)], fidx_v)

    def chunk_body(c, carry):
        crow = c * CR
        cb0 = c * CB

        pltpu.async_copy(ftab_hbm.at[fidx_v.at[pl.ds(cb0, CB)]], fbuf,
                         sem).wait()
        pltpu.sync_copy(fbuf, fout_hbm.at[pl.ds(bbase + cb0, CB)])

        handles = []
        for j in range(GSUB):
            handles.append(pltpu.async_copy(
                etab_hbm.at[eidx_v.at[pl.ds(crow + j * 128, 128)]],
                ebuf.at[pl.ds(j * 128, 128)], sem))
            handles.append(pltpu.async_copy(
                rtab_hbm.at[ridx_v.at[pl.ds(crow + j * 128, 128)]],
                rbuf.at[pl.ds(j * 128, 128)], sem))
        for h in handles:
            h.wait()

        def add_body(b, carry2):
            row0 = b * L
            for d in range(DIM // 16):
                fv = fbuf[b, pl.ds(d * 16, 16)]
                for l in range(L):
                    ebuf[row0 + l, pl.ds(d * 16, 16)] += fv
            return carry2

        lax.fori_loop(0, CB, add_body, 0)

        pltpu.sync_copy(ebuf, eout_hbm.at[pl.ds(rbase + crow, CR)])
        pltpu.sync_copy(rbuf, rout_hbm.at[pl.ds(rbase + crow, CR)])
        return carry

    lax.fori_loop(0, NCHUNK, chunk_body, 0)


@jax.jit
def _encode(element_ids, relation_ids, frame_id, element_embed,
            relation_embed, frame_embed):
    mesh = plsc.VectorSubcoreMesh(core_axis_name="c", subcore_axis_name="s",
                                  num_cores=NC, num_subcores=NS)
    f32 = jnp.float32
    run = functools.partial(
        pl.kernel,
        out_type=(
            jax.ShapeDtypeStruct((B * L, DIM), f32),
            jax.ShapeDtypeStruct((B * L, DIM), f32),
            jax.ShapeDtypeStruct((B, DIM), f32),
        ),
        mesh=mesh,
        scratch_types=[
            pltpu.VMEM((RW,), jnp.int32),
            pltpu.VMEM((RW,), jnp.int32),
            pltpu.VMEM((BW,), jnp.int32),
            pltpu.VMEM((CR, DIM), f32),
            pltpu.VMEM((CR, DIM), f32),
            pltpu.VMEM((CB, DIM), f32),
            pltpu.SemaphoreType.DMA,
        ],
    )(_sc_kernel)
    return run(element_ids.reshape(B * L), relation_ids.reshape(B * L),
               frame_id, element_embed, relation_embed, frame_embed)


def kernel(element_ids, relation_ids, frame_id, element_embed,
           relation_embed, frame_embed):
    eflat, rflat, frame = _encode(element_ids, relation_ids, frame_id,
                                  element_embed, relation_embed, frame_embed)
    return (eflat.reshape(B, L, DIM), rflat.reshape(B, L, DIM), frame)


# speedup vs baseline: 1.0328x; 1.0328x over previous
import functools

import jax
import jax.numpy as jnp
from jax import lax
from jax.experimental import pallas as pl
from jax.experimental.pallas import tpu as pltpu
from jax.experimental.pallas import tpu_sc as plsc

VOCAB = 1000000
DIM = 64
B = 16384
L = 20

NC = 2
NS = 16
NW = NC * NS

BW = B // NW
CB = 32
NCHUNK = BW // CB
RW = BW * L
CR = CB * L
GSUB = CR // 128


def _sc_kernel(eids_hbm, rids_hbm, fids_hbm, etab_hbm, rtab_hbm, ftab_hbm,
               eout_hbm, rout_hbm, fout_hbm,
               eidx_v, ridx_v, fidx_v, ebuf, rbuf, fbuf, sem):
    wid = lax.axis_index("s") * NC + lax.axis_index("c")
    rbase = wid * RW
    bbase = wid * BW

    pltpu.sync_copy(eids_hbm.at[pl.ds(rbase, RW)], eidx_v)
    pltpu.sync_copy(rids_hbm.at[pl.ds(rbase, RW)], ridx_v)
    pltpu.sync_copy(fids_hbm.at[pl.ds(bbase, BW)], fidx_v)

    def chunk_body(c, carry):
        crow = c * CR
        cb0 = c * CB

        pltpu.async_copy(ftab_hbm.at[fidx_v.at[pl.ds(cb0, CB)]], fbuf,
                         sem).wait()
        pltpu.sync_copy(fbuf, fout_hbm.at[pl.ds(bbase + cb0, CB)])

        handles = []
        for j in range(GSUB):
            handles.append(pltpu.async_copy(
                etab_hbm.at[eidx_v.at[pl.ds(crow + j * 128, 128)]],
                ebuf.at[pl.ds(j * 128, 128)], sem))
            handles.append(pltpu.async_copy(
                rtab_hbm.at[ridx_v.at[pl.ds(crow + j * 128, 128)]],
                rbuf.at[pl.ds(j * 128, 128)], sem))
        for h in handles:
            h.wait()

        def add_body(b, carry2):
            row0 = b * L
            for d in range(DIM // 16):
                fv = fbuf[b, pl.ds(d * 16, 16)]
                for l in range(L):
                    ebuf[row0 + l, pl.ds(d * 16, 16)] += fv
            return carry2

        lax.fori_loop(0, CB, add_body, 0)

        pltpu.sync_copy(ebuf, eout_hbm.at[pl.ds(rbase + crow, CR)])
        pltpu.sync_copy(rbuf, rout_hbm.at[pl.ds(rbase + crow, CR)])
        return carry

    lax.fori_loop(0, NCHUNK, chunk_body, 0)


@jax.jit
def _encode(element_ids, relation_ids, frame_id, element_embed,
            relation_embed, frame_embed):
    mesh = plsc.VectorSubcoreMesh(core_axis_name="c", subcore_axis_name="s",
                                  num_cores=NC, num_subcores=NS)
    f32 = jnp.float32
    run = functools.partial(
        pl.kernel,
        out_type=(
            jax.ShapeDtypeStruct((B * L, DIM), f32),
            jax.ShapeDtypeStruct((B * L, DIM), f32),
            jax.ShapeDtypeStruct((B, DIM), f32),
        ),
        mesh=mesh,
        compiler_params=pltpu.CompilerParams(use_tc_tiling_on_sc=False),
        scratch_types=[
            pltpu.VMEM((RW,), jnp.int32),
            pltpu.VMEM((RW,), jnp.int32),
            pltpu.VMEM((BW,), jnp.int32),
            pltpu.VMEM((CR, DIM), f32),
            pltpu.VMEM((CR, DIM), f32),
            pltpu.VMEM((CB, DIM), f32),
            pltpu.SemaphoreType.DMA,
        ],
    )(_sc_kernel)
    return run(element_ids.reshape(B * L), relation_ids.reshape(B * L),
               frame_id, element_embed, relation_embed, frame_embed)


def kernel(element_ids, relation_ids, frame_id, element_embed,
           relation_embed, frame_embed):
    eflat, rflat, frame = _encode(element_ids, relation_ids, frame_id,
                                  element_embed, relation_embed, frame_embed)
    return (eflat.reshape(B, L, DIM), rflat.reshape(B, L, DIM), frame)


# speedup vs baseline: 1.0334x; 1.0005x over previous
import functools

import jax
import jax.numpy as jnp
from jax import lax
from jax.experimental import pallas as pl
from jax.experimental.pallas import tpu as pltpu
from jax.experimental.pallas import tpu_sc as plsc

VOCAB = 1000000
DIM = 64
B = 16384
L = 20

NC = 2
NS = 16
NW = NC * NS

BW = B // NW
CB = 16
NCHUNK = BW // CB
RW = BW * L
CR = CB * L
GSUB = ((0, 128), (128, 128), (256, 64))


def _sc_kernel(eids_hbm, rids_hbm, fids_hbm, etab_hbm, rtab_hbm, ftab_hbm,
               eout_hbm, rout_hbm, fout_hbm,
               eidx_v, ridx_v, fidx_v, ebuf, rbuf, fbuf,
               gsem0, gsem1, osem0, osem1):
    wid = lax.axis_index("s") * NC + lax.axis_index("c")
    rbase = wid * RW
    bbase = wid * BW
    gsem = (gsem0, gsem1)
    osem = (osem0, osem1)

    pltpu.sync_copy(eids_hbm.at[pl.ds(rbase, RW)], eidx_v)
    pltpu.sync_copy(rids_hbm.at[pl.ds(rbase, RW)], ridx_v)
    pltpu.sync_copy(fids_hbm.at[pl.ds(bbase, BW)], fidx_v)

    def gather_descs(c, s):
        crow = c * CR
        cb0 = c * CB
        descs = [pltpu.make_async_copy(
            ftab_hbm.at[fidx_v.at[pl.ds(cb0, CB)]], fbuf.at[s], gsem[s])]
        for off, n in GSUB:
            descs.append(pltpu.make_async_copy(
                etab_hbm.at[eidx_v.at[pl.ds(crow + off, n)]],
                ebuf.at[s].at[pl.ds(off, n)], gsem[s]))
            descs.append(pltpu.make_async_copy(
                rtab_hbm.at[ridx_v.at[pl.ds(crow + off, n)]],
                rbuf.at[s].at[pl.ds(off, n)], gsem[s]))
        return descs

    def out_descs(c, s):
        crow = c * CR
        cb0 = c * CB
        return [
            pltpu.make_async_copy(fbuf.at[s],
                                  fout_hbm.at[pl.ds(bbase + cb0, CB)],
                                  osem[s]),
            pltpu.make_async_copy(ebuf.at[s],
                                  eout_hbm.at[pl.ds(rbase + crow, CR)],
                                  osem[s]),
            pltpu.make_async_copy(rbuf.at[s],
                                  rout_hbm.at[pl.ds(rbase + crow, CR)],
                                  osem[s]),
        ]

    def add_frame(s):
        def add_body(b, carry2):
            row0 = b * L
            for d in range(DIM // 16):
                fv = fbuf.at[s][b, pl.ds(d * 16, 16)]
                for l in range(L):
                    ebuf.at[s][row0 + l, pl.ds(d * 16, 16)] += fv
            return carry2

        lax.fori_loop(0, CB, add_body, 0)

    for d in gather_descs(0, 0):
        d.start()

    def pair_body(p, carry):
        for s in (0, 1):
            c = p * 2 + s
            s2 = 1 - s

            @pl.when(c >= 1)
            def _():
                for d in out_descs(c - 1, s2):
                    d.wait()

            @pl.when(c + 1 < NCHUNK)
            def _():
                for d in gather_descs(c + 1, s2):
                    d.start()

            for d in gather_descs(c, s):
                d.wait()
            add_frame(s)
            for d in out_descs(c, s):
                d.start()
        return carry

    lax.fori_loop(0, NCHUNK // 2, pair_body, 0)
    for d in out_descs(NCHUNK - 1, 1):
        d.wait()


@jax.jit
def _encode(element_ids, relation_ids, frame_id, element_embed,
            relation_embed, frame_embed):
    mesh = plsc.VectorSubcoreMesh(core_axis_name="c", subcore_axis_name="s",
                                  num_cores=NC, num_subcores=NS)
    f32 = jnp.float32
    run = functools.partial(
        pl.kernel,
        out_type=(
            jax.ShapeDtypeStruct((B * L, DIM), f32),
            jax.ShapeDtypeStruct((B * L, DIM), f32),
            jax.ShapeDtypeStruct((B, DIM), f32),
        ),
        mesh=mesh,
        compiler_params=pltpu.CompilerParams(use_tc_tiling_on_sc=False),
        scratch_types=[
            pltpu.VMEM((RW,), jnp.int32),
            pltpu.VMEM((RW,), jnp.int32),
            pltpu.VMEM((BW,), jnp.int32),
            pltpu.VMEM((2, CR, DIM), f32),
            pltpu.VMEM((2, CR, DIM), f32),
            pltpu.VMEM((2, CB, DIM), f32),
            pltpu.SemaphoreType.DMA,
            pltpu.SemaphoreType.DMA,
            pltpu.SemaphoreType.DMA,
            pltpu.SemaphoreType.DMA,
        ],
    )(_sc_kernel)
    return run(element_ids.reshape(B * L), relation_ids.reshape(B * L),
               frame_id, element_embed, relation_embed, frame_embed)


def kernel(element_ids, relation_ids, frame_id, element_embed,
           relation_embed, frame_embed):
    eflat, rflat, frame = _encode(element_ids, relation_ids, frame_id,
                                  element_embed, relation_embed, frame_embed)
    return (eflat.reshape(B, L, DIM), rflat.reshape(B, L, DIM), frame)


# speedup vs baseline: 1.0675x; 1.0330x over previous
import functools

import jax
import jax.numpy as jnp
from jax import lax
from jax.experimental import pallas as pl
from jax.experimental.pallas import tpu as pltpu
from jax.experimental.pallas import tpu_sc as plsc

VOCAB = 1000000
DIM = 64
B = 16384
L = 20

NC = 2
NS = 16
NW = NC * NS

BW = B // NW
CB = 16
NCHUNK = BW // CB
RW = BW * L
CR = CB * L
GSUB = ((0, 128), (128, 128), (256, 64))


def _sc_kernel(eids_hbm, rids_hbm, fids_hbm, etab_hbm, rtab_hbm, ftab_hbm,
               eout_hbm, rout_hbm, fout_hbm,
               eidx_v, ridx_v, fidx_v, ebuf, rbuf, fbuf, rtab_v, ftab_v,
               gsem0, gsem1, osem0, osem1):
    sid = lax.axis_index("s")
    wid = sid * NC + lax.axis_index("c")
    rbase = wid * RW
    bbase = wid * BW
    gsem = (gsem0, gsem1)
    osem = (osem0, osem1)

    pltpu.sync_copy(rtab_hbm, rtab_v)
    pltpu.sync_copy(ftab_hbm, ftab_v)

    pltpu.sync_copy(eids_hbm.at[pl.ds(rbase, RW)], eidx_v)
    pltpu.sync_copy(rids_hbm.at[pl.ds(rbase, RW)], ridx_v)
    pltpu.sync_copy(fids_hbm.at[pl.ds(bbase, BW)], fidx_v)

    def gather_descs(c, s):
        crow = c * CR
        return [pltpu.make_async_copy(
            etab_hbm.at[eidx_v.at[pl.ds(crow + off, n)]],
            ebuf.at[s].at[pl.ds(off, n)], gsem[s]) for off, n in GSUB]

    iota16 = lax.iota(jnp.int32, 16)

    def build_small(c, s):
        cb0 = c * CB
        crow = c * CR
        fid16 = fidx_v[pl.ds(cb0, CB)]
        for d in range(DIM):
            colv = jnp.full((16,), d, jnp.int32)
            v = plsc.load_gather(ftab_v, [fid16, colv])
            plsc.store_scatter(fbuf.at[s], [iota16, colv], v)

        def grp(g, carry):
            rid16 = ridx_v[pl.ds(crow + g * 16, 16)]
            rowv = iota16 + g * 16
            for d in range(DIM):
                colv = jnp.full((16,), d, jnp.int32)
                v = plsc.load_gather(rtab_v, [rid16, colv])
                plsc.store_scatter(rbuf.at[s], [rowv, colv], v)
            return carry

        lax.fori_loop(0, CR // 16, grp, 0)

    def out_descs(c, s):
        crow = c * CR
        cb0 = c * CB
        return [
            pltpu.make_async_copy(fbuf.at[s],
                                  fout_hbm.at[pl.ds(bbase + cb0, CB)],
                                  osem[s]),
            pltpu.make_async_copy(ebuf.at[s],
                                  eout_hbm.at[pl.ds(rbase + crow, CR)],
                                  osem[s]),
            pltpu.make_async_copy(rbuf.at[s],
                                  rout_hbm.at[pl.ds(rbase + crow, CR)],
                                  osem[s]),
        ]

    def add_frame(s):
        def add_body(b, carry2):
            row0 = b * L
            for d in range(DIM // 16):
                fv = fbuf.at[s][b, pl.ds(d * 16, 16)]
                for l in range(L):
                    ebuf.at[s][row0 + l, pl.ds(d * 16, 16)] += fv
            return carry2

        lax.fori_loop(0, CB, add_body, 0)

    for d in gather_descs(0, 0):
        d.start()

    def pair_body(p, carry):
        for s in (0, 1):
            c = p * 2 + s
            s2 = 1 - s

            @pl.when(c >= 1)
            def _():
                for d in out_descs(c - 1, s2):
                    d.wait()

            @pl.when(c + 1 < NCHUNK)
            def _():
                for d in gather_descs(c + 1, s2):
                    d.start()

            build_small(c, s)
            for d in gather_descs(c, s):
                d.wait()
            add_frame(s)
            for d in out_descs(c, s):
                d.start()
        return carry

    lax.fori_loop(0, NCHUNK // 2, pair_body, 0)
    for d in out_descs(NCHUNK - 1, 1):
        d.wait()


@jax.jit
def _encode(element_ids, relation_ids, frame_id, element_embed,
            relation_embed, frame_embed):
    mesh = plsc.VectorSubcoreMesh(core_axis_name="c", subcore_axis_name="s",
                                  num_cores=NC, num_subcores=NS)
    f32 = jnp.float32
    run = functools.partial(
        pl.kernel,
        out_type=(
            jax.ShapeDtypeStruct((B * L, DIM), f32),
            jax.ShapeDtypeStruct((B * L, DIM), f32),
            jax.ShapeDtypeStruct((B, DIM), f32),
        ),
        mesh=mesh,
        compiler_params=pltpu.CompilerParams(use_tc_tiling_on_sc=False,
                                             needs_layout_passes=False),
        scratch_types=[
            pltpu.VMEM((RW,), jnp.int32),
            pltpu.VMEM((RW,), jnp.int32),
            pltpu.VMEM((BW,), jnp.int32),
            pltpu.VMEM((2, CR, DIM), f32),
            pltpu.VMEM((2, CR, DIM), f32),
            pltpu.VMEM((2, CB, DIM), f32),
            pltpu.VMEM((20, DIM), f32),
            pltpu.VMEM((100, DIM), f32),
            pltpu.SemaphoreType.DMA,
            pltpu.SemaphoreType.DMA,
            pltpu.SemaphoreType.DMA,
            pltpu.SemaphoreType.DMA,
        ],
    )(_sc_kernel)
    return run(element_ids.reshape(B * L), relation_ids.reshape(B * L),
               frame_id, element_embed, relation_embed, frame_embed)


def kernel(element_ids, relation_ids, frame_id, element_embed,
           relation_embed, frame_embed):
    eflat, rflat, frame = _encode(element_ids, relation_ids, frame_id,
                                  element_embed, relation_embed, frame_embed)
    return (eflat.reshape(B, L, DIM), rflat.reshape(B, L, DIM), frame)
